# Initial kernel scaffold; baseline (speedup 1.0000x reference)
import jax
import jax.numpy as jnp
from jax.experimental import pallas as pl


def kernel(x, gating_kernel, routed_kernel_0, routed_bias_0, shared_kernel_0, shared_bias_0, head_kernel, head_bias):
    raise NotImplementedError("write your pallas kernel here")


# speedup vs baseline: 8.9033x; 8.9033x over previous
import functools

import jax
import jax.numpy as jnp
from jax import lax
from jax.experimental import pallas as pl
from jax.experimental.pallas import tpu as pltpu

B = 1024
IN_DIM = 768
NUM_TASKS = 8
NUM_EXPERTS = 64
NUM_SHARED = 2
WIDTH = 128
HEAD_DIM = 32


def _moe_body(feats_ref, task_ref, gk_ref, rk_ref, rb_ref, sk_ref, sb_ref,
              hk_ref, hb_ref, out_ref,
              acc_ref, i1_ref, i2_ref, w1_ref, tid_ref):
    e = pl.program_id(0)

    @pl.when(e == 0)
    def _init():
        task = task_ref[...]
        logits = jnp.dot(task, gk_ref[...], preferred_element_type=jnp.float32)
        iota = lax.broadcasted_iota(jnp.int32, (B, NUM_EXPERTS), 1)
        m1 = jnp.max(logits, axis=1, keepdims=True)
        i1 = jnp.min(jnp.where(logits == m1, iota, NUM_EXPERTS), axis=1,
                     keepdims=True)
        l2 = jnp.where(iota == i1, -jnp.inf, logits)
        m2 = jnp.max(l2, axis=1, keepdims=True)
        i2 = jnp.min(jnp.where(l2 == m2, iota, NUM_EXPERTS), axis=1,
                     keepdims=True)
        w1 = 1.0 / (1.0 + jnp.exp(m2 - m1))
        i1_ref[...] = i1
        i2_ref[...] = i2
        w1_ref[...] = w1
        t_iota = lax.broadcasted_iota(jnp.int32, (B, NUM_TASKS), 1)
        tmax = jnp.max(task, axis=1, keepdims=True)
        tid_ref[...] = jnp.min(jnp.where(task == tmax, t_iota, NUM_TASKS),
                               axis=1, keepdims=True)
        feats = feats_ref[...]
        s = jnp.zeros((B, WIDTH), jnp.float32)
        for j in range(NUM_SHARED):
            h = jnp.dot(feats, sk_ref[j], preferred_element_type=jnp.float32)
            s = s + jax.nn.relu(h + sb_ref[j][None, :])
        acc_ref[...] = s * (1.0 / NUM_SHARED)

    we = (jnp.where(i1_ref[...] == e, w1_ref[...], 0.0)
          + jnp.where(i2_ref[...] == e, 1.0 - w1_ref[...], 0.0))
    h = jnp.dot(feats_ref[...], rk_ref[0], preferred_element_type=jnp.float32)
    h = jax.nn.relu(h + rb_ref[pl.ds(e, 1), :])
    acc_ref[...] += we * h

    @pl.when(e == NUM_EXPERTS - 1)
    def _final():
        f = jnp.tanh(acc_ref[...])
        heads = jnp.dot(f, hk_ref[...], preferred_element_type=jnp.float32)
        heads = heads + hb_ref[...]
        cols = lax.broadcasted_iota(jnp.int32, (B, NUM_TASKS * HEAD_DIM), 1)
        sel = jnp.where(cols // HEAD_DIM == tid_ref[...], heads, 0.0)
        fold = (lax.broadcasted_iota(jnp.int32, (NUM_TASKS * HEAD_DIM, HEAD_DIM), 0) % HEAD_DIM
                == lax.broadcasted_iota(jnp.int32, (NUM_TASKS * HEAD_DIM, HEAD_DIM), 1)
                ).astype(jnp.float32)
        out_ref[...] = jnp.dot(sel, fold, preferred_element_type=jnp.float32)


@jax.jit
def kernel(x, gating_kernel, routed_kernel_0, routed_bias_0,
           shared_kernel_0, shared_bias_0, head_kernel, head_bias):
    feats = x[:, :IN_DIM]
    task = x[:, IN_DIM:]
    hk2 = head_kernel.transpose(1, 0, 2).reshape(WIDTH, NUM_TASKS * HEAD_DIM)
    hb2 = head_bias.reshape(1, NUM_TASKS * HEAD_DIM)

    full = lambda shape: pl.BlockSpec(shape, lambda e: (0,) * len(shape))
    grid_spec = pltpu.PrefetchScalarGridSpec(
        num_scalar_prefetch=0,
        grid=(NUM_EXPERTS,),
        in_specs=[
            full((B, IN_DIM)),
            full((B, NUM_TASKS)),
            full((NUM_TASKS, NUM_EXPERTS)),
            pl.BlockSpec((1, IN_DIM, WIDTH), lambda e: (e, 0, 0)),
            full((NUM_EXPERTS, WIDTH)),
            full((NUM_SHARED, IN_DIM, WIDTH)),
            full((NUM_SHARED, WIDTH)),
            full((WIDTH, NUM_TASKS * HEAD_DIM)),
            full((1, NUM_TASKS * HEAD_DIM)),
        ],
        out_specs=full((B, HEAD_DIM)),
        scratch_shapes=[
            pltpu.VMEM((B, WIDTH), jnp.float32),
            pltpu.VMEM((B, 1), jnp.int32),
            pltpu.VMEM((B, 1), jnp.int32),
            pltpu.VMEM((B, 1), jnp.float32),
            pltpu.VMEM((B, 1), jnp.int32),
        ],
    )
    return pl.pallas_call(
        _moe_body,
        grid_spec=grid_spec,
        out_shape=jax.ShapeDtypeStruct((B, HEAD_DIM), jnp.float32),
        compiler_params=pltpu.CompilerParams(
            dimension_semantics=("arbitrary",)),
    )(feats, task, gating_kernel, routed_kernel_0, routed_bias_0,
      shared_kernel_0, shared_bias_0, hk2, hb2)
